# baseline (speedup 1.0000x reference)
import dataclasses
import functools

import jax
import jax.numpy as jnp
from jax import lax
from jax.experimental import pallas as pl
from jax.experimental.pallas import tpu as pltpu
from jax.experimental.pallas import tpu_sc as plsc

N_NODES = 10000
N_EDGES = 320000
D_FEAT = 128
D_OUT = 2
NC = 2
NS = 16
NW = NC * NS
L = 16
FLAT = N_NODES * D_OUT
CH = (N_EDGES // NW) // 128 * 128
REM = N_EDGES - NW * CH
CHBUF = CH + REM


def _mesh():
    return plsc.VectorSubcoreMesh(core_axis_name="c", subcore_axis_name="s")


def _sc_params():
    cp = pltpu.CompilerParams()
    if "needs_layout_passes" in pltpu.CompilerParams.__dataclass_fields__:
        cp = dataclasses.replace(cp, needs_layout_passes=False)
    return cp


def _sc_degree_partials(edges):
    @functools.partial(
        pl.kernel,
        out_type=jax.ShapeDtypeStruct((2 * NW * N_NODES,), jnp.float32),
        mesh=_mesh(),
        compiler_params=_sc_params(),
        scratch_types=[
            pltpu.VMEM((2, CHBUF), jnp.int32),
            pltpu.VMEM((N_NODES,), jnp.float32),
            pltpu.VMEM((N_NODES,), jnp.float32),
        ],
    )
    def hist_kernel(edge_hbm, out_hbm, edge_v, dego_v, degi_v):
        wid = lax.axis_index("c") * NS + lax.axis_index("s")
        base = wid * CH
        pltpu.sync_copy(edge_hbm.at[:, pl.ds(base, CH)], edge_v.at[:, pl.ds(0, CH)])
        last = wid == NW - 1

        @pl.when(last)
        def _():
            pltpu.sync_copy(edge_hbm.at[:, pl.ds(NW * CH, REM)],
                            edge_v.at[:, pl.ds(CH, REM)])

        zf = jnp.zeros((L,), jnp.float32)

        @pl.loop(0, N_NODES, step=L, unroll=8)
        def _(i):
            dego_v[pl.ds(i, L)] = zf
            degi_v[pl.ds(i, L)] = zf

        onesf = jnp.ones((L,), jnp.float32)

        def step(i):
            s16 = edge_v[0, pl.ds(i, L)]
            d16 = edge_v[1, pl.ds(i, L)]
            plsc.addupdate_scatter(dego_v, [s16], onesf)
            plsc.addupdate_scatter(degi_v, [d16], onesf)

        @plsc.parallel_loop(0, CH, step=L, unroll=8)
        def _(i):
            step(i)

        @pl.when(last)
        def _():
            @pl.loop(CH, CHBUF, step=L, unroll=8)
            def _(i):
                step(i)

        pltpu.sync_copy(dego_v, out_hbm.at[pl.ds(wid * N_NODES, N_NODES)])
        pltpu.sync_copy(degi_v, out_hbm.at[pl.ds((NW + wid) * N_NODES, N_NODES)])

    return hist_kernel(edges)


def _sc_aggregate_partials(feat_flat, edges):
    @functools.partial(
        pl.kernel,
        out_type=jax.ShapeDtypeStruct((2 * NW * N_NODES,), jnp.float32),
        mesh=_mesh(),
        compiler_params=_sc_params(),
        scratch_types=[
            pltpu.VMEM((2, CHBUF), jnp.int32),
            pltpu.VMEM((FLAT,), jnp.float32),
            pltpu.VMEM((FLAT,), jnp.float32),
        ],
    )
    def agg_kernel(feat_hbm, edge_hbm, out_hbm, edge_v, feat_v, acc_v):
        wid = lax.axis_index("c") * NS + lax.axis_index("s")
        base = wid * CH
        pltpu.sync_copy(feat_hbm, feat_v)
        pltpu.sync_copy(edge_hbm.at[:, pl.ds(base, CH)], edge_v.at[:, pl.ds(0, CH)])
        last = wid == NW - 1

        @pl.when(last)
        def _():
            pltpu.sync_copy(edge_hbm.at[:, pl.ds(NW * CH, REM)],
                            edge_v.at[:, pl.ds(CH, REM)])

        zf = jnp.zeros((L,), jnp.float32)

        @pl.loop(0, FLAT, step=L, unroll=8)
        def _(i):
            acc_v[pl.ds(i, L)] = zf

        offk = jnp.full((L,), N_NODES, jnp.int32)

        def step(i):
            s16 = edge_v[0, pl.ds(i, L)]
            d16 = edge_v[1, pl.ds(i, L)]
            v0 = plsc.load_gather(feat_v, [s16])
            v1 = plsc.load_gather(feat_v, [s16 + offk])
            plsc.addupdate_scatter(acc_v, [d16], v0)
            plsc.addupdate_scatter(acc_v, [d16 + offk], v1)

        @plsc.parallel_loop(0, CH, step=L, unroll=8)
        def _(i):
            step(i)

        @pl.when(last)
        def _():
            @pl.loop(CH, CHBUF, step=L, unroll=8)
            def _(i):
                step(i)

        pltpu.sync_copy(acc_v.at[pl.ds(0, N_NODES)],
                        out_hbm.at[pl.ds(wid * N_NODES, N_NODES)])
        pltpu.sync_copy(acc_v.at[pl.ds(N_NODES, N_NODES)],
                        out_hbm.at[pl.ds((NW + wid) * N_NODES, N_NODES)])

    return agg_kernel(feat_flat, edges)


def _tc_project(x, W):
    def body(x_ref, w_ref, y_ref):
        y_ref[...] = jax.lax.dot_general(
            w_ref[...], x_ref[...],
            dimension_numbers=(((0,), (1,)), ((), ())),
            preferred_element_type=jnp.float32,
            precision=jax.lax.Precision.HIGHEST,
        )

    return pl.pallas_call(
        body,
        out_shape=jax.ShapeDtypeStruct((D_OUT, N_NODES), jnp.float32),
    )(x, W)


def _tc_norm_feat(deg_part, yT):
    def body(dp_ref, y_ref, feat_ref, nd_ref):
        dego = dp_ref[pl.ds(0, N_NODES)]
        degi = dp_ref[pl.ds(NW * N_NODES, N_NODES)]
        for w in range(1, NW):
            dego = dego + dp_ref[pl.ds(w * N_NODES, N_NODES)]
            degi = degi + dp_ref[pl.ds((NW + w) * N_NODES, N_NODES)]
        do = jnp.maximum(dego, 1.0)
        di = jnp.maximum(degi, 1.0)
        ro = jax.lax.rsqrt(do)
        ri = jax.lax.rsqrt(di)
        ns = ro * (1.5 - 0.5 * do * ro * ro)
        nd_ref[...] = ri * (1.5 - 0.5 * di * ri * ri)
        y = y_ref[...]
        feat_ref[pl.ds(0, N_NODES)] = y[0, :] * ns
        feat_ref[pl.ds(N_NODES, N_NODES)] = y[1, :] * ns

    return pl.pallas_call(
        body,
        out_shape=(
            jax.ShapeDtypeStruct((FLAT,), jnp.float32),
            jax.ShapeDtypeStruct((N_NODES,), jnp.float32),
        ),
    )(deg_part, yT)


def _tc_finish(agg_part, nd, b):
    def body(ap_ref, nd_ref, b_ref, o_ref):
        a0 = ap_ref[pl.ds(0, N_NODES)]
        a1 = ap_ref[pl.ds(NW * N_NODES, N_NODES)]
        for w in range(1, NW):
            a0 = a0 + ap_ref[pl.ds(w * N_NODES, N_NODES)]
            a1 = a1 + ap_ref[pl.ds((NW + w) * N_NODES, N_NODES)]
        nd = nd_ref[...]
        out = jnp.stack([a0 * nd + b_ref[0], a1 * nd + b_ref[1]], axis=0)
        o_ref[...] = out.T

    return pl.pallas_call(
        body,
        out_shape=jax.ShapeDtypeStruct((N_NODES, D_OUT), jnp.float32),
    )(agg_part, nd, b)


def kernel(x, edge_index, W, b):
    edges = edge_index.astype(jnp.int32)
    yT = _tc_project(x, W)
    deg_part = _sc_degree_partials(edges)
    feat_flat, nd = _tc_norm_feat(deg_part, yT)
    agg_part = _sc_aggregate_partials(feat_flat, edges)
    return _tc_finish(agg_part, nd, b)


# speedup vs baseline: 1.1562x; 1.1562x over previous
import dataclasses
import functools

import jax
import jax.numpy as jnp
from jax import lax
from jax.experimental import pallas as pl
from jax.experimental.pallas import tpu as pltpu
from jax.experimental.pallas import tpu_sc as plsc

N_NODES = 10000
N_EDGES = 320000
D_FEAT = 128
D_OUT = 2
NC = 2
NS = 16
NW = NC * NS
L = 16
FLAT = N_NODES * D_OUT
CH = (N_EDGES // NW) // 128 * 128
REM = N_EDGES - NW * CH
CHBUF = CH + REM


def _mesh():
    return plsc.VectorSubcoreMesh(core_axis_name="c", subcore_axis_name="s")


def _sc_params():
    cp = pltpu.CompilerParams()
    if "needs_layout_passes" in pltpu.CompilerParams.__dataclass_fields__:
        cp = dataclasses.replace(cp, needs_layout_passes=False)
    return cp


def _sc_degree_partials(edges):
    @functools.partial(
        pl.kernel,
        out_type=jax.ShapeDtypeStruct((2 * NW * N_NODES,), jnp.float32),
        mesh=_mesh(),
        compiler_params=_sc_params(),
        scratch_types=[
            pltpu.VMEM((2, CHBUF), jnp.int32),
            pltpu.VMEM((N_NODES,), jnp.float32),
            pltpu.VMEM((N_NODES,), jnp.float32),
        ],
    )
    def hist_kernel(edge_hbm, out_hbm, edge_v, dego_v, degi_v):
        wid = lax.axis_index("c") * NS + lax.axis_index("s")
        base = wid * CH
        pltpu.sync_copy(edge_hbm.at[:, pl.ds(base, CH)], edge_v.at[:, pl.ds(0, CH)])
        last = wid == NW - 1

        @pl.when(last)
        def _():
            pltpu.sync_copy(edge_hbm.at[:, pl.ds(NW * CH, REM)],
                            edge_v.at[:, pl.ds(CH, REM)])

        zf = jnp.zeros((L,), jnp.float32)

        @pl.loop(0, N_NODES, step=L, unroll=8)
        def _(i):
            dego_v[pl.ds(i, L)] = zf
            degi_v[pl.ds(i, L)] = zf

        onesf = jnp.ones((L,), jnp.float32)

        def step(i):
            s16 = edge_v[0, pl.ds(i, L)]
            d16 = edge_v[1, pl.ds(i, L)]
            plsc.addupdate_scatter(dego_v, [s16], onesf)
            plsc.addupdate_scatter(degi_v, [d16], onesf)

        @plsc.parallel_loop(0, CH, step=L, unroll=8)
        def _(i):
            step(i)

        @pl.when(last)
        def _():
            @pl.loop(CH, CHBUF, step=L, unroll=8)
            def _(i):
                step(i)

        pltpu.sync_copy(dego_v, out_hbm.at[pl.ds(wid * N_NODES, N_NODES)])
        pltpu.sync_copy(degi_v, out_hbm.at[pl.ds((NW + wid) * N_NODES, N_NODES)])

    return hist_kernel(edges)


def _sc_aggregate_partials(feat_flat, edges):
    @functools.partial(
        pl.kernel,
        out_type=jax.ShapeDtypeStruct((2 * NW * N_NODES,), jnp.float32),
        mesh=_mesh(),
        compiler_params=_sc_params(),
        scratch_types=[
            pltpu.VMEM((2, CHBUF), jnp.int32),
            pltpu.VMEM((FLAT,), jnp.float32),
            pltpu.VMEM((FLAT,), jnp.float32),
        ],
    )
    def agg_kernel(feat_hbm, edge_hbm, out_hbm, edge_v, feat_v, acc_v):
        wid = lax.axis_index("c") * NS + lax.axis_index("s")
        base = wid * CH
        pltpu.sync_copy(feat_hbm, feat_v)
        pltpu.sync_copy(edge_hbm.at[:, pl.ds(base, CH)], edge_v.at[:, pl.ds(0, CH)])
        last = wid == NW - 1

        @pl.when(last)
        def _():
            pltpu.sync_copy(edge_hbm.at[:, pl.ds(NW * CH, REM)],
                            edge_v.at[:, pl.ds(CH, REM)])

        zf = jnp.zeros((L,), jnp.float32)

        @pl.loop(0, FLAT, step=L, unroll=8)
        def _(i):
            acc_v[pl.ds(i, L)] = zf

        offk = jnp.full((L,), N_NODES, jnp.int32)

        def step(i):
            s16 = edge_v[0, pl.ds(i, L)]
            d16 = edge_v[1, pl.ds(i, L)]
            v0 = plsc.load_gather(feat_v, [s16])
            v1 = plsc.load_gather(feat_v, [s16 + offk])
            plsc.addupdate_scatter(acc_v, [d16], v0)
            plsc.addupdate_scatter(acc_v, [d16 + offk], v1)

        @plsc.parallel_loop(0, CH, step=L, unroll=8)
        def _(i):
            step(i)

        @pl.when(last)
        def _():
            @pl.loop(CH, CHBUF, step=L, unroll=8)
            def _(i):
                step(i)

        pltpu.sync_copy(acc_v.at[pl.ds(0, N_NODES)],
                        out_hbm.at[pl.ds(wid * N_NODES, N_NODES)])
        pltpu.sync_copy(acc_v.at[pl.ds(N_NODES, N_NODES)],
                        out_hbm.at[pl.ds((NW + wid) * N_NODES, N_NODES)])

    return agg_kernel(feat_flat, edges)


def _tc_project(x, W):
    def body(x_ref, w_ref, y_ref):
        y_ref[...] = jax.lax.dot_general(
            w_ref[...], x_ref[...],
            dimension_numbers=(((0,), (1,)), ((), ())),
            preferred_element_type=jnp.float32,
            precision=jax.lax.Precision.HIGHEST,
        )

    return pl.pallas_call(
        body,
        out_shape=jax.ShapeDtypeStruct((D_OUT, N_NODES), jnp.float32),
    )(x, W)


def _tc_norm_feat(deg_part, yT):
    def body(dp_ref, y_ref, feat_ref, nd_ref):
        dego = dp_ref[pl.ds(0, N_NODES)]
        degi = dp_ref[pl.ds(NW * N_NODES, N_NODES)]
        for w in range(1, NW):
            dego = dego + dp_ref[pl.ds(w * N_NODES, N_NODES)]
            degi = degi + dp_ref[pl.ds((NW + w) * N_NODES, N_NODES)]
        do = jnp.maximum(dego, 1.0)
        di = jnp.maximum(degi, 1.0)
        ro = jax.lax.rsqrt(do)
        ri = jax.lax.rsqrt(di)
        ns = ro * (1.5 - 0.5 * do * ro * ro)
        nd_ref[...] = ri * (1.5 - 0.5 * di * ri * ri)
        y = y_ref[...]
        feat_ref[pl.ds(0, N_NODES)] = y[0, :] * ns
        feat_ref[pl.ds(N_NODES, N_NODES)] = y[1, :] * ns

    return pl.pallas_call(
        body,
        out_shape=(
            jax.ShapeDtypeStruct((FLAT,), jnp.float32),
            jax.ShapeDtypeStruct((N_NODES,), jnp.float32),
        ),
    )(deg_part, yT)


def _tc_finish(agg_part, nd, b):
    def body(ap_ref, nd_ref, b_ref, o_ref):
        a0 = ap_ref[pl.ds(0, N_NODES)]
        a1 = ap_ref[pl.ds(NW * N_NODES, N_NODES)]
        for w in range(1, NW):
            a0 = a0 + ap_ref[pl.ds(w * N_NODES, N_NODES)]
            a1 = a1 + ap_ref[pl.ds((NW + w) * N_NODES, N_NODES)]
        nd = nd_ref[...]
        o_ref[0, :] = a0 * nd + b_ref[0]
        o_ref[1, :] = a1 * nd + b_ref[1]

    return pl.pallas_call(
        body,
        out_shape=jax.ShapeDtypeStruct((D_OUT, N_NODES), jnp.float32),
    )(agg_part, nd, b)


def kernel(x, edge_index, W, b):
    edges = edge_index.astype(jnp.int32)
    yT = _tc_project(x, W)
    deg_part = _sc_degree_partials(edges)
    feat_flat, nd = _tc_norm_feat(deg_part, yT)
    agg_part = _sc_aggregate_partials(feat_flat, edges)
    return _tc_finish(agg_part, nd, b).T
